# baseline (speedup 1.0000x reference)
import jax
import jax.numpy as jnp
from jax.experimental import pallas as pl
from jax.experimental.pallas import tpu as pltpu

Q, K, D = 4096, 16384, 512
TQ = 512
NQ = Q // TQ
KCHUNK = 16384
NKC = K // KCHUNK
LANES = 128


def _knn_kernel(qt_ref, kb_ref, ps_ref, img_ref, ksqh_ref):
    i = pl.program_id(0)

    @pl.when(i == 0)
    def _ksq():
        kbf = kb_ref[...].astype(jnp.float32)
        ksqh_ref[...] = 0.5 * jnp.sum(kbf * kbf, axis=1, keepdims=True)

    qt8 = qt_ref[...]
    m = None
    for c in range(NKC):
        rows = pl.ds(c * KCHUNK, KCHUNK)
        cross = jax.lax.dot_general(
            kb_ref[rows, :], qt8,
            (((1,), (0,)), ((), ())),
            preferred_element_type=jnp.float32)
        s = cross - ksqh_ref[rows, :]
        pm = jnp.max(s, axis=0)
        m = pm if m is None else jnp.maximum(m, pm)

    qtf = qt8.astype(jnp.float32)
    q_sq = jnp.sum(qtf * qtf, axis=0)
    d2 = q_sq - 2.0 * m
    scores = jnp.sqrt(jnp.maximum(d2, 0.0) + 1e-12)
    ps_ref[...] = scores
    bmax = jnp.max(scores)

    @pl.when(i == 0)
    def _img_init():
        img_ref[...] = jnp.broadcast_to(bmax, (1, LANES))

    @pl.when(i > 0)
    def _img_fold():
        img_ref[...] = jnp.maximum(img_ref[...], bmax)


def kernel(query_features, memory_bank):
    qt8 = query_features.T.astype(jnp.float8_e4m3fn)
    kb8 = memory_bank.astype(jnp.float8_e4m3fn)
    patch_scores, img = pl.pallas_call(
        _knn_kernel,
        grid=(NQ,),
        in_specs=[
            pl.BlockSpec((D, TQ), lambda i: (0, i)),
            pl.BlockSpec((K, D), lambda i: (0, 0)),
        ],
        out_specs=[
            pl.BlockSpec((TQ,), lambda i: (i,)),
            pl.BlockSpec((1, LANES), lambda i: (0, 0)),
        ],
        out_shape=[
            jax.ShapeDtypeStruct((Q,), jnp.float32),
            jax.ShapeDtypeStruct((1, LANES), jnp.float32),
        ],
        scratch_shapes=[
            pltpu.VMEM((K, 1), jnp.float32),
        ],
    )(qt8, kb8)
    return patch_scores, img[0, :1]


# speedup vs baseline: 1.2133x; 1.2133x over previous
import jax
import jax.numpy as jnp
from jax.experimental import pallas as pl
from jax.experimental.pallas import tpu as pltpu

Q, K, D = 4096, 16384, 512
TQ = 512
NQ = Q // TQ
KCHUNK = 8192
NKC = K // KCHUNK
LANES = 128


def _knn_kernel(qt_ref, kb_ref, ps_ref, img_ref, ksqh_ref):
    i = pl.program_id(0)

    @pl.when(i == 0)
    def _ksq():
        kbf = kb_ref[...].astype(jnp.float32)
        ksqh_ref[...] = 0.5 * jnp.sum(kbf * kbf, axis=1, keepdims=True)

    q8 = qt_ref[...]
    m = None
    for c in range(NKC):
        rows = pl.ds(c * KCHUNK, KCHUNK)
        cross = jax.lax.dot_general(
            kb_ref[rows, :], q8,
            (((1,), (1,)), ((), ())),
            preferred_element_type=jnp.float32)
        s = cross - ksqh_ref[rows, :]
        pm = jnp.max(s, axis=0)
        m = pm if m is None else jnp.maximum(m, pm)

    qtf = q8.astype(jnp.float32)
    q_sq = jnp.sum(qtf * qtf, axis=1)
    d2 = q_sq - 2.0 * m
    scores = jnp.sqrt(jnp.maximum(d2, 0.0) + 1e-12)
    ps_ref[...] = scores
    bmax = jnp.max(scores)

    @pl.when(i == 0)
    def _img_init():
        img_ref[...] = jnp.broadcast_to(bmax, (1, LANES))

    @pl.when(i > 0)
    def _img_fold():
        img_ref[...] = jnp.maximum(img_ref[...], bmax)


def kernel(query_features, memory_bank):
    q8 = query_features.astype(jnp.float8_e4m3fn)
    kb8 = memory_bank.astype(jnp.float8_e4m3fn)
    patch_scores, img = pl.pallas_call(
        _knn_kernel,
        grid=(NQ,),
        in_specs=[
            pl.BlockSpec((TQ, D), lambda i: (i, 0)),
            pl.BlockSpec((K, D), lambda i: (0, 0)),
        ],
        out_specs=[
            pl.BlockSpec((TQ,), lambda i: (i,)),
            pl.BlockSpec((1, LANES), lambda i: (0, 0)),
        ],
        out_shape=[
            jax.ShapeDtypeStruct((Q,), jnp.float32),
            jax.ShapeDtypeStruct((1, LANES), jnp.float32),
        ],
        scratch_shapes=[
            pltpu.VMEM((K, 1), jnp.float32),
        ],
    )(q8, kb8)
    return patch_scores, img[0, :1]


# speedup vs baseline: 1.4595x; 1.2029x over previous
import jax
import jax.numpy as jnp
from jax.experimental import pallas as pl
from jax.experimental.pallas import tpu as pltpu

Q, K, D = 4096, 16384, 512
TQ = 512
NQ = Q // TQ
KW = K // NQ
KCHUNK = 8192
NKC = K // KCHUNK
LANES = 128


def _knn_kernel(q_ref, kbw_ref, ps_ref, img_ref, kb8_ref, ksqh_ref):
    p = pl.program_id(0)
    i = pl.program_id(1)

    @pl.when(p == 0)
    def _convert():
        kbf = kbw_ref[...]
        rows = pl.ds(i * KW, KW)
        kb8_ref[rows, :] = kbf.astype(jnp.float8_e4m3fn)
        ksqh_ref[rows, :] = 0.5 * jnp.sum(kbf * kbf, axis=1, keepdims=True)

    @pl.when(p == 1)
    def _score():
        qf = q_ref[...]
        q8 = qf.astype(jnp.float8_e4m3fn)
        m = None
        for c in range(NKC):
            rows = pl.ds(c * KCHUNK, KCHUNK)
            cross = jax.lax.dot_general(
                kb8_ref[rows, :], q8,
                (((1,), (1,)), ((), ())),
                preferred_element_type=jnp.float32)
            s = cross - ksqh_ref[rows, :]
            pm = jnp.max(s, axis=0)
            m = pm if m is None else jnp.maximum(m, pm)

        q_sq = jnp.sum(qf * qf, axis=1)
        d2 = q_sq - 2.0 * m
        scores = jnp.sqrt(jnp.maximum(d2, 0.0) + 1e-12)
        ps_ref[...] = scores
        bmax = jnp.max(scores)

        @pl.when(i == 0)
        def _img_init():
            img_ref[...] = jnp.broadcast_to(bmax, (1, LANES))

        @pl.when(i > 0)
        def _img_fold():
            img_ref[...] = jnp.maximum(img_ref[...], bmax)


def kernel(query_features, memory_bank):
    patch_scores, img = pl.pallas_call(
        _knn_kernel,
        grid=(2, NQ),
        in_specs=[
            pl.BlockSpec((TQ, D), lambda p, i: (i * p, 0)),
            pl.BlockSpec((KW, D), lambda p, i: (i * (1 - p), 0)),
        ],
        out_specs=[
            pl.BlockSpec((TQ,), lambda p, i: (i,)),
            pl.BlockSpec((1, LANES), lambda p, i: (0, 0)),
        ],
        out_shape=[
            jax.ShapeDtypeStruct((Q,), jnp.float32),
            jax.ShapeDtypeStruct((1, LANES), jnp.float32),
        ],
        scratch_shapes=[
            pltpu.VMEM((K, D), jnp.float8_e4m3fn),
            pltpu.VMEM((K, 1), jnp.float32),
        ],
    )(query_features, memory_bank)
    return patch_scores, img[0, :1]
